# Initial kernel scaffold; baseline (speedup 1.0000x reference)
import jax
import jax.numpy as jnp
from jax.experimental import pallas as pl


def kernel(words, chars, table):
    raise NotImplementedError("write your pallas kernel here")


# speedup vs baseline: 3.2936x; 3.2936x over previous
import jax
import jax.numpy as jnp
from jax import lax
from jax.experimental import pallas as pl
from jax.experimental.pallas import tpu as pltpu
from jax.experimental.pallas import tpu_sc as plsc

BATCH = 1024
MAX_WORDS = 50
MAX_CHARS = 20
EMBED_DIM = 64
VOCAB = 1001

NC, NS, L = 2, 16, 16
NW = NC * NS
TOTAL_WORDS = BATCH * MAX_WORDS
WPT = TOTAL_WORDS // NW
CHUNK = 400
NCHUNK = WPT // CHUNK


def _sc_body(chars_hbm, table_hbm, out_hbm, table_v, chars_v, out_v):
    wid = lax.axis_index("s") * NC + lax.axis_index("c")
    pltpu.sync_copy(table_hbm, table_v)
    iota = lax.iota(jnp.int32, L)
    word0 = wid * WPT
    for k in range(NCHUNK):
        cbase = word0 + k * CHUNK
        pltpu.sync_copy(
            chars_hbm.at[pl.ds(cbase * MAX_CHARS, CHUNK * MAX_CHARS)], chars_v
        )

        def group_body(g, _):
            gw = g * L
            cvs = [
                plsc.load_gather(
                    chars_v, [iota * MAX_CHARS + (gw * MAX_CHARS + c)]
                )
                for c in range(MAX_CHARS)
            ]
            rows = [cv * EMBED_DIM for cv in cvs]

            def d_body(d, _):
                acc = plsc.load_gather(table_v, [rows[0] + d])
                for c in range(1, MAX_CHARS):
                    acc = jnp.maximum(
                        acc, plsc.load_gather(table_v, [rows[c] + d])
                    )
                plsc.store_scatter(
                    out_v, [iota * EMBED_DIM + (gw * EMBED_DIM + d)], acc
                )
                return 0

            lax.fori_loop(0, EMBED_DIM, d_body, 0, unroll=2)
            return 0

        lax.fori_loop(0, CHUNK // L, group_body, 0)
        pltpu.sync_copy(
            out_v, out_hbm.at[pl.ds(cbase * EMBED_DIM, CHUNK * EMBED_DIM)]
        )


def kernel(words, chars, table):
    del words
    chars_flat = chars.reshape(-1).astype(jnp.int32)
    table_flat = table.reshape(-1)
    mesh = plsc.VectorSubcoreMesh(core_axis_name="c", subcore_axis_name="s")
    run = pl.kernel(
        _sc_body,
        out_type=jax.ShapeDtypeStruct((TOTAL_WORDS * EMBED_DIM,), jnp.float32),
        mesh=mesh,
        scratch_types=[
            pltpu.VMEM((VOCAB * EMBED_DIM,), jnp.float32),
            pltpu.VMEM((CHUNK * MAX_CHARS,), jnp.int32),
            pltpu.VMEM((CHUNK * EMBED_DIM,), jnp.float32),
        ],
        compiler_params=pltpu.CompilerParams(needs_layout_passes=False),
    )
    out = run(chars_flat, table_flat)
    return out.reshape(BATCH, MAX_WORDS, EMBED_DIM)


# speedup vs baseline: 4.3836x; 1.3310x over previous
import jax
import jax.numpy as jnp
from jax import lax
from jax.experimental import pallas as pl
from jax.experimental.pallas import tpu as pltpu
from jax.experimental.pallas import tpu_sc as plsc

BATCH = 1024
MAX_WORDS = 50
MAX_CHARS = 20
EMBED_DIM = 64
VOCAB = 1001

NC, NS, L = 2, 16, 16
NW = NC * NS
TOTAL_WORDS = BATCH * MAX_WORDS
WPT = TOTAL_WORDS // NW
CHUNK = 400
NCHUNK = WPT // CHUNK


def _sc_body(chars_hbm, table_hbm, out_hbm, table_v, chars_v, out_v):
    wid = lax.axis_index("s") * NC + lax.axis_index("c")
    pltpu.sync_copy(table_hbm, table_v)
    iota = lax.iota(jnp.int32, L)
    word0 = wid * WPT
    for k in range(NCHUNK):
        cbase = word0 + k * CHUNK
        pltpu.sync_copy(
            chars_hbm.at[pl.ds(cbase * MAX_CHARS, CHUNK * MAX_CHARS)], chars_v
        )

        DB = 8

        def group_body(g, _):
            gw = g * L
            cvs = [
                plsc.load_gather(
                    chars_v, [iota * MAX_CHARS + (gw * MAX_CHARS + c)]
                )
                for c in range(MAX_CHARS)
            ]
            rows = [cv * EMBED_DIM for cv in cvs]
            obase = iota * EMBED_DIM + gw * EMBED_DIM

            def block_body(blk, _):
                dbase = blk * DB
                accs = [
                    plsc.load_gather(table_v, [rows[0] + (dbase + j)])
                    for j in range(DB)
                ]
                for c in range(1, MAX_CHARS):
                    for j in range(DB):
                        accs[j] = jnp.maximum(
                            accs[j],
                            plsc.load_gather(table_v, [rows[c] + (dbase + j)]),
                        )
                for j in range(DB):
                    plsc.store_scatter(out_v, [obase + (dbase + j)], accs[j])
                return 0

            lax.fori_loop(0, EMBED_DIM // DB, block_body, 0)
            return 0

        lax.fori_loop(0, CHUNK // L, group_body, 0)
        pltpu.sync_copy(
            out_v, out_hbm.at[pl.ds(cbase * EMBED_DIM, CHUNK * EMBED_DIM)]
        )


def kernel(words, chars, table):
    del words
    chars_flat = chars.reshape(-1).astype(jnp.int32)
    table_flat = table.reshape(-1)
    mesh = plsc.VectorSubcoreMesh(core_axis_name="c", subcore_axis_name="s")
    run = pl.kernel(
        _sc_body,
        out_type=jax.ShapeDtypeStruct((TOTAL_WORDS * EMBED_DIM,), jnp.float32),
        mesh=mesh,
        scratch_types=[
            pltpu.VMEM((VOCAB * EMBED_DIM,), jnp.float32),
            pltpu.VMEM((CHUNK * MAX_CHARS,), jnp.int32),
            pltpu.VMEM((CHUNK * EMBED_DIM,), jnp.float32),
        ],
        compiler_params=pltpu.CompilerParams(needs_layout_passes=False),
    )
    out = run(chars_flat, table_flat)
    return out.reshape(BATCH, MAX_WORDS, EMBED_DIM)
